# baseline (device time: 66654 ns/iter reference)
import jax
import jax.numpy as jnp
from jax import lax
from jax.experimental import pallas as pl
from jax.experimental.pallas import tpu as pltpu

N_DEV = 8

PERMS = ((1, 3, 4), (3, 4, 1), (4, 1, 3))
P_OFF = (0, 688, 1376)
P_LEN = (688, 688, 672)

EXCH = {
    "e0": (0, (), True),
    "e1": (1, (), True),
    "e2": (1, (0,), True),
    "e3": (2, (), False),
    "e4": (2, (0,), True),
    "e5": (2, (1,), True),
    "e6": (2, (0, 1), True),
}
N_SEMS = 3 * (2 * 6 + 1)


def kernel(x):
    m_per, n = x.shape
    assert m_per == P_OFF[2] + P_LEN[2]

    def body(x_ref, out_ref, send_sems, recv_sems):
        my = lax.axis_index("i")

        barrier_sem = pltpu.get_barrier_semaphore()
        for mask in (1, 3, 4):
            pl.semaphore_signal(
                barrier_sem, inc=1,
                device_id=(my ^ mask,), device_id_type=pl.DeviceIdType.MESH,
            )
        pl.semaphore_wait(barrier_sem, 3)

        send_d = {}
        recv_d = {}
        sem_idx = 0
        for s, perm in enumerate(PERMS):
            off, ln = P_OFF[s], P_LEN[s]
            for tag, (pi, ci, halved) in EXCH.items():
                pmask = perm[pi]
                cmask = 0
                for i in ci:
                    cmask ^= perm[i]
                send_chunk = my ^ cmask
                recv_chunk = my ^ cmask ^ pmask
                halves = ((0, ln // 2), (ln // 2, ln - ln // 2)) if halved \
                    else ((0, ln),)
                for h, (hoff, hlen) in enumerate(halves):
                    def region(chunk, o=off + hoff, l=hlen):
                        return out_ref.at[pl.ds(chunk * m_per + o, l), :]

                    common = dict(
                        send_sem=send_sems.at[sem_idx],
                        recv_sem=recv_sems.at[sem_idx],
                        device_id=(my ^ pmask,),
                        device_id_type=pl.DeviceIdType.MESH,
                    )
                    sem_idx += 1
                    send_d[s, tag, h] = pltpu.make_async_remote_copy(
                        src_ref=region(send_chunk),
                        dst_ref=region(send_chunk),
                        **common,
                    )
                    recv_d[s, tag, h] = pltpu.make_async_remote_copy(
                        src_ref=region(recv_chunk),
                        dst_ref=region(recv_chunk),
                        **common,
                    )
        assert sem_idx == N_SEMS

        for s in range(3):
            off, ln = P_OFF[s], P_LEN[s]
            out_ref[pl.ds(my * m_per + off, ln), :] = (
                x_ref[off:off + ln, :].astype(out_ref.dtype)
            )
            send_d[s, "e0", 0].start()
            send_d[s, "e0", 1].start()
        for s in range(3):
            send_d[s, "e1", 0].start()
            send_d[s, "e1", 1].start()
        for s in range(3):
            send_d[s, "e3", 0].start()
        for h in range(2):
            for s in range(3):
                recv_d[s, "e0", h].wait_recv()
                send_d[s, "e2", h].start()
                send_d[s, "e4", h].start()
        for h in range(2):
            for s in range(3):
                recv_d[s, "e1", h].wait_recv()
                send_d[s, "e5", h].start()
        for h in range(2):
            for s in range(3):
                recv_d[s, "e2", h].wait_recv()
                send_d[s, "e6", h].start()
        for s in range(3):
            recv_d[s, "e3", 0].wait_recv()
            for tag in ("e4", "e5", "e6"):
                for h in range(2):
                    recv_d[s, tag, h].wait_recv()
        for key, d in send_d.items():
            d.wait_send()

    out_shape = jax.ShapeDtypeStruct((N_DEV * m_per, n), jnp.bfloat16)
    return pl.pallas_call(
        body,
        out_shape=out_shape,
        in_specs=[pl.BlockSpec(memory_space=pltpu.VMEM)],
        out_specs=pl.BlockSpec(memory_space=pltpu.VMEM),
        scratch_shapes=[
            pltpu.SemaphoreType.DMA((N_SEMS,)),
            pltpu.SemaphoreType.DMA((N_SEMS,)),
        ],
        compiler_params=pltpu.CompilerParams(collective_id=0),
    )(x)


# device time: 66498 ns/iter; 1.0023x vs baseline; 1.0023x over previous
import jax
import jax.numpy as jnp
from jax import lax
from jax.experimental import pallas as pl
from jax.experimental.pallas import tpu as pltpu

N_DEV = 8

PERMS = ((1, 3, 4), (3, 4, 1), (4, 1, 3))
P_OFF = (0, 688, 1376)
P_LEN = (688, 688, 672)

EXCH = {
    "e0": (0, (), True),
    "e1": (1, (), True),
    "e2": (1, (0,), True),
    "e3": (2, (), False),
    "e4": (2, (0,), True),
    "e5": (2, (1,), True),
    "e6": (2, (0, 1), True),
}
N_SEMS = 3 * (2 * 6 + 1)


def kernel(x):
    m_per, n = x.shape
    assert m_per == P_OFF[2] + P_LEN[2]

    def body(x_ref, out_ref, send_sems, recv_sems):
        my = lax.axis_index("i")

        barrier_sem = pltpu.get_barrier_semaphore()
        for mask in (1, 3, 4):
            pl.semaphore_signal(
                barrier_sem, inc=1,
                device_id=(my ^ mask,), device_id_type=pl.DeviceIdType.MESH,
            )
        pl.semaphore_wait(barrier_sem, 3)

        send_d = {}
        recv_d = {}
        sem_idx = 0
        for s, perm in enumerate(PERMS):
            off, ln = P_OFF[s], P_LEN[s]
            for tag, (pi, ci, halved) in EXCH.items():
                pmask = perm[pi]
                cmask = 0
                for i in ci:
                    cmask ^= perm[i]
                send_chunk = my ^ cmask
                recv_chunk = my ^ cmask ^ pmask
                halves = ((0, ln // 2), (ln // 2, ln - ln // 2)) if halved \
                    else ((0, ln),)
                for h, (hoff, hlen) in enumerate(halves):
                    def region(chunk, o=off + hoff, l=hlen):
                        return out_ref.at[pl.ds(chunk * m_per + o, l), :]

                    common = dict(
                        send_sem=send_sems.at[sem_idx],
                        recv_sem=recv_sems.at[sem_idx],
                        device_id=(my ^ pmask,),
                        device_id_type=pl.DeviceIdType.MESH,
                    )
                    sem_idx += 1
                    send_d[s, tag, h] = pltpu.make_async_remote_copy(
                        src_ref=region(send_chunk),
                        dst_ref=region(send_chunk),
                        **common,
                    )
                    recv_d[s, tag, h] = pltpu.make_async_remote_copy(
                        src_ref=region(recv_chunk),
                        dst_ref=region(recv_chunk),
                        **common,
                    )
        assert sem_idx == N_SEMS

        order = (2, 0, 1)
        for s in order:
            off, ln = P_OFF[s], P_LEN[s]
            out_ref[pl.ds(my * m_per + off, ln), :] = (
                x_ref[off:off + ln, :].astype(out_ref.dtype)
            )
            send_d[s, "e0", 0].start()
            send_d[s, "e0", 1].start()
        for s in order:
            send_d[s, "e1", 0].start()
            send_d[s, "e1", 1].start()
        for s in order:
            send_d[s, "e3", 0].start()
        for h in range(2):
            for s in range(3):
                recv_d[s, "e0", h].wait_recv()
                send_d[s, "e2", h].start()
                send_d[s, "e4", h].start()
        for h in range(2):
            for s in range(3):
                recv_d[s, "e1", h].wait_recv()
                send_d[s, "e5", h].start()
        for h in range(2):
            for s in range(3):
                recv_d[s, "e2", h].wait_recv()
                send_d[s, "e6", h].start()
        for s in range(3):
            recv_d[s, "e3", 0].wait_recv()
            for tag in ("e4", "e5", "e6"):
                for h in range(2):
                    recv_d[s, tag, h].wait_recv()
        for key, d in send_d.items():
            d.wait_send()

    out_shape = jax.ShapeDtypeStruct((N_DEV * m_per, n), jnp.bfloat16)
    return pl.pallas_call(
        body,
        out_shape=out_shape,
        in_specs=[pl.BlockSpec(memory_space=pltpu.VMEM)],
        out_specs=pl.BlockSpec(memory_space=pltpu.VMEM),
        scratch_shapes=[
            pltpu.SemaphoreType.DMA((N_SEMS,)),
            pltpu.SemaphoreType.DMA((N_SEMS,)),
        ],
        compiler_params=pltpu.CompilerParams(collective_id=0),
    )(x)


# device time: 65098 ns/iter; 1.0239x vs baseline; 1.0215x over previous
import jax
import jax.numpy as jnp
from jax import lax
from jax.experimental import pallas as pl
from jax.experimental.pallas import tpu as pltpu

N_DEV = 8

PERMS = ((1, 3, 4), (3, 4, 1), (4, 1, 3))
E_PARTNER = (0, 1, 1, 2, 2, 2, 2)
E_CHUNK = ((), (), (0,), (), (0,), (1,), (0, 1))
N_EXCH = 7

P_OFF = (0, 688, 1376)
P_LEN = (688, 688, 672)


def kernel(x):
    m_per, n = x.shape
    assert m_per == P_OFF[2] + P_LEN[2]

    def body(x_ref, out_ref, send_sems, recv_sems):
        my = lax.axis_index("i")

        out_ref[pl.ds(my * m_per, m_per), :] = x_ref[:, :].astype(out_ref.dtype)

        barrier_sem = pltpu.get_barrier_semaphore()
        for mask in (1, 3, 4):
            pl.semaphore_signal(
                barrier_sem, inc=1,
                device_id=(my ^ mask,), device_id_type=pl.DeviceIdType.MESH,
            )
        pl.semaphore_wait(barrier_sem, 3)

        send_d = {}
        recv_d = {}
        for s, perm in enumerate(PERMS):
            off, ln = P_OFF[s], P_LEN[s]

            def region(chunk, off=off, ln=ln):
                return out_ref.at[pl.ds(chunk * m_per + off, ln), :]

            for e in range(N_EXCH):
                pmask = perm[E_PARTNER[e]]
                cmask = 0
                for i in E_CHUNK[e]:
                    cmask ^= perm[i]
                k = s * N_EXCH + e
                send_chunk = my ^ cmask
                recv_chunk = my ^ cmask ^ pmask
                common = dict(
                    send_sem=send_sems.at[k],
                    recv_sem=recv_sems.at[k],
                    device_id=(my ^ pmask,),
                    device_id_type=pl.DeviceIdType.MESH,
                )
                send_d[s, e] = pltpu.make_async_remote_copy(
                    src_ref=region(send_chunk), dst_ref=region(send_chunk),
                    **common,
                )
                recv_d[s, e] = pltpu.make_async_remote_copy(
                    src_ref=region(recv_chunk), dst_ref=region(recv_chunk),
                    **common,
                )

        for s in range(3):
            send_d[s, 0].start()
        for s in range(3):
            send_d[s, 1].start()
        for s in range(3):
            send_d[s, 3].start()
        for s in range(3):
            recv_d[s, 0].wait_recv()
            send_d[s, 2].start()
            send_d[s, 4].start()
        for s in range(3):
            recv_d[s, 1].wait_recv()
            send_d[s, 5].start()
        for s in range(3):
            recv_d[s, 2].wait_recv()
            send_d[s, 6].start()
        for s in range(3):
            for e in (3, 4, 5, 6):
                recv_d[s, e].wait_recv()
        for s in range(3):
            for e in range(N_EXCH):
                send_d[s, e].wait_send()

    out_shape = jax.ShapeDtypeStruct((N_DEV * m_per, n), jnp.bfloat16)
    return pl.pallas_call(
        body,
        out_shape=out_shape,
        in_specs=[pl.BlockSpec(memory_space=pltpu.VMEM)],
        out_specs=pl.BlockSpec(memory_space=pltpu.VMEM),
        scratch_shapes=[
            pltpu.SemaphoreType.DMA((3 * N_EXCH,)),
            pltpu.SemaphoreType.DMA((3 * N_EXCH,)),
        ],
        compiler_params=pltpu.CompilerParams(collective_id=0),
    )(x)
